# baseline (speedup 1.0000x reference)
import functools

import jax
import jax.numpy as jnp
from jax import lax
from jax.experimental import pallas as pl
from jax.experimental.pallas import tpu as pltpu
from jax.experimental.pallas import tpu_sc as plsc

BATCH = 4
SEQ = 8192
EMB = 1024
NUM_CORES = 2
NUM_SUBCORES = 16
NUM_WORKERS = NUM_CORES * NUM_SUBCORES
ROWS_PER_WORKER = SEQ // NUM_WORKERS

_mesh = plsc.VectorSubcoreMesh(core_axis_name="c", subcore_axis_name="s")


@functools.partial(
    pl.kernel,
    mesh=_mesh,
    out_type=jax.ShapeDtypeStruct((BATCH, SEQ, EMB), jnp.float32),
    scratch_types=[pltpu.SemaphoreType.DMA],
)
def _broadcast_table(table_hbm, out_hbm, sem):
    wid = lax.axis_index("s") * NUM_CORES + lax.axis_index("c")
    base = wid * ROWS_PER_WORKER
    src = table_hbm.at[pl.ds(base, ROWS_PER_WORKER)]
    copies = [
        pltpu.make_async_copy(src, out_hbm.at[b, pl.ds(base, ROWS_PER_WORKER)], sem)
        for b in range(BATCH)
    ]
    for c in copies:
        c.start()
    for c in copies:
        c.wait()


def kernel(src, seg, table):
    del src, seg
    return _broadcast_table(table)


# speedup vs baseline: 53.7274x; 53.7274x over previous
import functools

import jax
import jax.numpy as jnp
from jax import lax
from jax.experimental import pallas as pl
from jax.experimental.pallas import tpu as pltpu
from jax.experimental.pallas import tpu_sc as plsc

BATCH = 4
SEQ = 8192
EMB = 1024
NUM_CORES = 2
NUM_SUBCORES = 16
NUM_WORKERS = NUM_CORES * NUM_SUBCORES
ROWS_PER_WORKER = SEQ // NUM_WORKERS
CHUNK_ROWS = 32
NUM_CHUNKS = ROWS_PER_WORKER // CHUNK_ROWS

_mesh = plsc.VectorSubcoreMesh(core_axis_name="c", subcore_axis_name="s")


@functools.partial(
    pl.kernel,
    mesh=_mesh,
    out_type=jax.ShapeDtypeStruct((BATCH, SEQ, EMB), jnp.float32),
    scratch_types=[
        pltpu.VMEM((CHUNK_ROWS, EMB), jnp.float32),
        pltpu.VMEM((CHUNK_ROWS, EMB), jnp.float32),
        pltpu.SemaphoreType.DMA,
        pltpu.SemaphoreType.DMA,
        pltpu.SemaphoreType.DMA,
    ],
)
def _broadcast_table(table_hbm, out_hbm, buf0, buf1, gsem, ssem0, ssem1):
    wid = lax.axis_index("s") * NUM_CORES + lax.axis_index("c")
    base = wid * ROWS_PER_WORKER
    bufs = (buf0, buf1)
    ssems = (ssem0, ssem1)

    def gather(i):
        return pltpu.make_async_copy(
            table_hbm.at[pl.ds(base + i * CHUNK_ROWS, CHUNK_ROWS)],
            bufs[i % 2],
            gsem,
        )

    def scatters(i):
        return [
            pltpu.make_async_copy(
                bufs[i % 2],
                out_hbm.at[b, pl.ds(base + i * CHUNK_ROWS, CHUNK_ROWS)],
                ssems[i % 2],
            )
            for b in range(BATCH)
        ]

    g = gather(0)
    g.start()
    pending = {}
    for i in range(NUM_CHUNKS):
        g.wait()
        if i + 1 < NUM_CHUNKS:
            if i - 1 in pending:
                for h in pending.pop(i - 1):
                    h.wait()
            g = gather(i + 1)
            g.start()
        sc = scatters(i)
        for h in sc:
            h.start()
        pending[i] = sc
    for i in sorted(pending):
        for h in pending[i]:
            h.wait()


def kernel(src, seg, table):
    del src, seg
    return _broadcast_table(table)


# speedup vs baseline: 77.5700x; 1.4438x over previous
import functools

import jax
import jax.numpy as jnp
from jax import lax
from jax.experimental import pallas as pl
from jax.experimental.pallas import tpu as pltpu
from jax.experimental.pallas import tpu_sc as plsc

BATCH = 4
SEQ = 8192
EMB = 1024
BS = 512


def _tc_body(tab_ref, out_ref):
    t = tab_ref[...]
    for b in range(BATCH):
        out_ref[b] = t


_tc_call = pl.pallas_call(
    _tc_body,
    grid=(SEQ // BS,),
    in_specs=[pl.BlockSpec((BS, EMB), lambda i: (i, 0))],
    out_specs=pl.BlockSpec((BATCH, BS, EMB), lambda i: (0, i, 0)),
    out_shape=jax.ShapeDtypeStruct((BATCH, SEQ, EMB), jnp.float32),
)


def kernel(src, seg, table):
    del src, seg
    return _tc_call(table)
